# baseline (speedup 1.0000x reference)
import functools

import jax
import jax.numpy as jnp
from jax import lax
from jax.experimental import pallas as pl
from jax.experimental.pallas import tpu as pltpu
from jax.experimental.pallas import tpu_sc as plsc

N = 100000
E = 3200000
NPAD = 102400
RP = NPAD // 16
PW = 8
ROWS = 10
LANES = 128
CHUNK = ROWS * LANES
NCHUNKS = E // CHUNK
NW = 32
ITERS = (NCHUNKS + NW - 1) // NW

_f32 = jnp.float32
_i32 = jnp.int32


def _sc_aggregate(x_pair, zeros, edge4):
    mesh = plsc.VectorSubcoreMesh(core_axis_name="c", subcore_axis_name="s")

    @functools.partial(
        pl.kernel,
        out_type=jax.ShapeDtypeStruct((2, NPAD, PW), _f32),
        mesh=mesh,
        scratch_types=(
            [pltpu.VMEM((LANES,), _i32) for _ in range(4 * ROWS)]
            + [pltpu.VMEM((LANES, PW), _f32) for _ in range(2 * ROWS)]
            + [
                pltpu.VMEM_SHARED((NPAD, PW), _f32),
                pltpu.VMEM_SHARED((NPAD, PW), _f32),
                pltpu.SemaphoreType.DMA,
                pltpu.SemaphoreType.DMA,
                pltpu.SemaphoreType.DMA,
                pltpu.SemaphoreType.DMA,
                pltpu.SemaphoreType.DMA,
            ]
        ),
        compiler_params=pltpu.CompilerParams(use_tc_tiling_on_sc=False),
    )
    def sc_agg(xp_hbm, z_hbm, edge_hbm, out_hbm, *refs):
        src_a = refs[0:ROWS]
        dst_a = refs[ROWS:2 * ROWS]
        src_b = refs[2 * ROWS:3 * ROWS]
        dst_b = refs[3 * ROWS:4 * ROWS]
        val_a = refs[4 * ROWS:5 * ROWS]
        val_b = refs[5 * ROWS:6 * ROWS]
        x_sp, acc_sp, gsem, isem_a, isem_b, psem_a, psem_b = refs[6 * ROWS:]
        cid = lax.axis_index("c")
        sid = lax.axis_index("s")
        wid = sid * 2 + cid
        off = sid * RP

        pltpu.sync_copy(xp_hbm.at[pl.ds(off, RP)], x_sp.at[pl.ds(off, RP)])
        pltpu.sync_copy(z_hbm.at[pl.ds(off, RP)], acc_sp.at[pl.ds(off, RP)])

        plsc.subcore_barrier()

        def issue_idx(t, src_v, dst_v, isem):
            for r in range(ROWS):
                pltpu.async_copy(edge_hbm.at[0, t, r], src_v[r], isem)
            for r in range(ROWS):
                pltpu.async_copy(edge_hbm.at[1, t, r], dst_v[r], isem)

        def wait_idx(t, src_v, dst_v, isem):
            for r in range(ROWS):
                pltpu.make_async_copy(edge_hbm.at[0, t, r], src_v[r],
                                      isem).wait()
            for r in range(ROWS):
                pltpu.make_async_copy(edge_hbm.at[1, t, r], dst_v[r],
                                      isem).wait()

        def process(src_v, dst_v, val_v, psem):
            gets = [
                pltpu.async_copy(x_sp.at[src_v[r]], val_v[r], gsem)
                for r in range(ROWS)
            ]
            for r in range(ROWS):
                gets[r].wait()
                pltpu.async_copy(val_v[r], acc_sp.at[dst_v[r]],
                                 psem, add=True)

        def wait_puts(dst_v, val_v, psem):
            for r in range(ROWS):
                pltpu.make_async_copy(val_v[r], acc_sp.at[dst_v[r]],
                                      psem).wait()

        issue_idx(wid, src_a, dst_a, isem_a)

        def chunk_body(i, _):
            ta = wid + NW * (2 * i)
            tb = ta + NW
            ta2 = ta + 2 * NW

            @pl.when(ta < NCHUNKS)
            def _():
                wait_idx(ta, src_a, dst_a, isem_a)
                process(src_a, dst_a, val_a, psem_a)

            @pl.when(jnp.logical_and(i > 0, ta - NW < NCHUNKS))
            def _():
                wait_puts(dst_b, val_b, psem_b)

            @pl.when(tb < NCHUNKS)
            def _():
                issue_idx(tb, src_b, dst_b, isem_b)
                wait_idx(tb, src_b, dst_b, isem_b)
                process(src_b, dst_b, val_b, psem_b)

            @pl.when(ta < NCHUNKS)
            def _():
                wait_puts(dst_a, val_a, psem_a)

            @pl.when(ta2 < NCHUNKS)
            def _():
                issue_idx(ta2, src_a, dst_a, isem_a)

            return 0

        lax.fori_loop(0, (ITERS + 1) // 2, chunk_body, 0)

        plsc.subcore_barrier()

        pltpu.sync_copy(acc_sp.at[pl.ds(off, RP)],
                        out_hbm.at[cid, pl.ds(off, RP)])

    return sc_agg(x_pair, zeros, edge4)


def _epilogue(a0, a1, c0, c1, x2, wl, bl, wr, wlin, blin):
    def body(a0_r, a1_r, c0_r, c1_r, x_r, wl_r, bl_r, wr_r, wlin_r, blin_r, out_r):
        agg = a0_r[...] + a1_r[...]
        cnt = jnp.maximum(c0_r[...] + c1_r[...], 1.0)
        mean = agg / cnt
        xv = x_r[...]

        kk = lax.broadcasted_iota(_i32, (100, 400), 0)
        jj = lax.broadcasted_iota(_i32, (100, 400), 1)
        f = jj - 4 * (jj // 4)
        sel = (jj // 4) == kk

        def expand(w_r):
            v = jnp.where(
                f == 0, w_r[0, 0],
                jnp.where(f == 1, w_r[0, 1],
                          jnp.where(f == 2, w_r[0, 2], w_r[0, 3])))
            return jnp.where(sel, v, 0.0)

        s_l = expand(wl_r)
        s_r = expand(wr_r)

        j2 = lax.broadcasted_iota(_i32, (8, 400), 1)
        f2 = j2 - 4 * (j2 // 4)
        brow = jnp.where(
            f2 == 0, bl_r[0, 0],
            jnp.where(f2 == 1, bl_r[0, 1],
                      jnp.where(f2 == 2, bl_r[0, 2], bl_r[0, 3])))[:1]

        h = (jax.lax.dot(mean, s_l, precision=jax.lax.Precision.HIGHEST,
                         preferred_element_type=_f32)
             + jax.lax.dot(xv, s_r, precision=jax.lax.Precision.HIGHEST,
                           preferred_element_type=_f32))
        h = jnp.maximum(h + brow, 0.0)
        out = jax.lax.dot_general(
            h, wlin_r[...], (((1,), (1,)), ((), ())),
            precision=jax.lax.Precision.HIGHEST, preferred_element_type=_f32)
        out_r[...] = out + blin_r[...]

    return pl.pallas_call(
        body,
        out_shape=jax.ShapeDtypeStruct((1000, 10), _f32),
    )(a0, a1, c0, c1, x2, wl, bl, wr, wlin, blin)


def kernel(x, edge_index, W_l, b_l, W_r, W_lin, b_lin):
    x_pair = jnp.pad(
        jnp.concatenate([x, jnp.ones_like(x)], axis=1),
        ((0, NPAD - N), (0, PW - 2)))
    zeros = jnp.zeros((NPAD, PW), _f32)
    edge4 = edge_index.reshape(2, NCHUNKS, ROWS, LANES)
    accP = _sc_aggregate(x_pair, zeros, edge4)

    a0 = accP[0, :N, 0].reshape(1000, 100)
    a1 = accP[1, :N, 0].reshape(1000, 100)
    c0 = accP[0, :N, 1].reshape(1000, 100)
    c1 = accP[1, :N, 1].reshape(1000, 100)
    x2 = x.reshape(1000, 100)
    wl = W_l.reshape(1, 4)
    wr = W_r.reshape(1, 4)
    bl = b_l.reshape(1, 4)
    blin = b_lin.reshape(1, 10)
    return _epilogue(a0, a1, c0, c1, x2, wl, bl, wr, W_lin, blin)


# speedup vs baseline: 1.0710x; 1.0710x over previous
import functools

import jax
import jax.numpy as jnp
from jax import lax
from jax.experimental import pallas as pl
from jax.experimental.pallas import tpu as pltpu
from jax.experimental.pallas import tpu_sc as plsc

N = 100000
E = 3200000
NPAD = 102400
RP = NPAD // 16
PW = 8
ROWS = 8
LANES = 128
CHUNK = ROWS * LANES
NCHUNKS = E // CHUNK
NW = 32
ITERS = (NCHUNKS + NW - 1) // NW

_f32 = jnp.float32
_i32 = jnp.int32


def _sc_aggregate(x_pair, zeros, edge4):
    mesh = plsc.VectorSubcoreMesh(core_axis_name="c", subcore_axis_name="s")

    @functools.partial(
        pl.kernel,
        out_type=jax.ShapeDtypeStruct((2, NPAD, PW), _f32),
        mesh=mesh,
        scratch_types=(
            [pltpu.VMEM((LANES,), _i32) for _ in range(8 * ROWS)]
            + [pltpu.VMEM((LANES, PW), _f32) for _ in range(2 * ROWS)]
            + [
                pltpu.VMEM_SHARED((NPAD, PW), _f32),
                pltpu.VMEM_SHARED((NPAD, PW), _f32),
                pltpu.SemaphoreType.DMA,
                pltpu.SemaphoreType.DMA,
                pltpu.SemaphoreType.DMA,
                pltpu.SemaphoreType.DMA,
                pltpu.SemaphoreType.DMA,
                pltpu.SemaphoreType.DMA,
                pltpu.SemaphoreType.DMA,
            ]
        ),
        compiler_params=pltpu.CompilerParams(use_tc_tiling_on_sc=False),
    )
    def sc_agg(xp_hbm, z_hbm, edge_hbm, out_hbm, *refs):
        src_s = [refs[2 * s * ROWS:(2 * s + 1) * ROWS] for s in range(4)]
        dst_s = [refs[(2 * s + 1) * ROWS:(2 * s + 2) * ROWS] for s in range(4)]
        val_s = [refs[(8 + v) * ROWS:(9 + v) * ROWS] for v in range(2)]
        (x_sp, acc_sp, gsem, isem0, isem1, isem2, isem3,
         psem0, psem1) = refs[10 * ROWS:]
        isem_s = [isem0, isem1, isem2, isem3]
        psem_s = [psem0, psem1]
        cid = lax.axis_index("c")
        sid = lax.axis_index("s")
        wid = sid * 2 + cid
        off = sid * RP

        pltpu.sync_copy(xp_hbm.at[pl.ds(off, RP)], x_sp.at[pl.ds(off, RP)])
        pltpu.sync_copy(z_hbm.at[pl.ds(off, RP)], acc_sp.at[pl.ds(off, RP)])

        plsc.subcore_barrier()

        def issue_idx(t, src_v, dst_v, isem):
            for r in range(ROWS):
                pltpu.async_copy(edge_hbm.at[0, t, r], src_v[r], isem)
            for r in range(ROWS):
                pltpu.async_copy(edge_hbm.at[1, t, r], dst_v[r], isem)

        def wait_idx(t, src_v, dst_v, isem):
            for r in range(ROWS):
                pltpu.make_async_copy(edge_hbm.at[0, t, r], src_v[r],
                                      isem).wait()
            for r in range(ROWS):
                pltpu.make_async_copy(edge_hbm.at[1, t, r], dst_v[r],
                                      isem).wait()

        def process(src_v, dst_v, val_v, psem):
            gets = [
                pltpu.async_copy(x_sp.at[src_v[r]], val_v[r], gsem)
                for r in range(ROWS)
            ]
            for r in range(ROWS):
                gets[r].wait()
                pltpu.async_copy(val_v[r], acc_sp.at[dst_v[r]],
                                 psem, add=True)

        def wait_puts(dst_v, val_v, psem):
            for r in range(ROWS):
                pltpu.make_async_copy(val_v[r], acc_sp.at[dst_v[r]],
                                      psem).wait()

        issue_idx(wid, src_s[0], dst_s[0], isem_s[0])
        issue_idx(wid + NW, src_s[1], dst_s[1], isem_s[1])

        def chunk_body(i, _):
            for u in range(4):
                k = 4 * i + u
                t = wid + NW * k
                tp = t + 2 * NW

                @pl.when(t < NCHUNKS)
                def _(u=u, t=t):
                    wait_idx(t, src_s[u], dst_s[u], isem_s[u])
                    process(src_s[u], dst_s[u], val_s[u % 2], psem_s[u % 2])

                up = (u - 1) % 4

                @pl.when(jnp.logical_and(k > 0, t - NW < NCHUNKS))
                def _(up=up):
                    wait_puts(dst_s[up], val_s[up % 2], psem_s[up % 2])

                un = (u + 2) % 4

                @pl.when(tp < NCHUNKS)
                def _(un=un, tp=tp):
                    issue_idx(tp, src_s[un], dst_s[un], isem_s[un])

            return 0

        lax.fori_loop(0, (ITERS + 3) // 4, chunk_body, 0)

        plsc.subcore_barrier()

        pltpu.sync_copy(acc_sp.at[pl.ds(off, RP)],
                        out_hbm.at[cid, pl.ds(off, RP)])

    return sc_agg(x_pair, zeros, edge4)


def _epilogue(a0, a1, c0, c1, x2, wl, bl, wr, wlin, blin):
    def body(a0_r, a1_r, c0_r, c1_r, x_r, wl_r, bl_r, wr_r, wlin_r, blin_r, out_r):
        agg = a0_r[...] + a1_r[...]
        cnt = jnp.maximum(c0_r[...] + c1_r[...], 1.0)
        mean = agg / cnt
        xv = x_r[...]

        kk = lax.broadcasted_iota(_i32, (100, 400), 0)
        jj = lax.broadcasted_iota(_i32, (100, 400), 1)
        f = jj - 4 * (jj // 4)
        sel = (jj // 4) == kk

        def expand(w_r):
            v = jnp.where(
                f == 0, w_r[0, 0],
                jnp.where(f == 1, w_r[0, 1],
                          jnp.where(f == 2, w_r[0, 2], w_r[0, 3])))
            return jnp.where(sel, v, 0.0)

        s_l = expand(wl_r)
        s_r = expand(wr_r)

        j2 = lax.broadcasted_iota(_i32, (8, 400), 1)
        f2 = j2 - 4 * (j2 // 4)
        brow = jnp.where(
            f2 == 0, bl_r[0, 0],
            jnp.where(f2 == 1, bl_r[0, 1],
                      jnp.where(f2 == 2, bl_r[0, 2], bl_r[0, 3])))[:1]

        h = (jax.lax.dot(mean, s_l, precision=jax.lax.Precision.HIGHEST,
                         preferred_element_type=_f32)
             + jax.lax.dot(xv, s_r, precision=jax.lax.Precision.HIGHEST,
                           preferred_element_type=_f32))
        h = jnp.maximum(h + brow, 0.0)
        out = jax.lax.dot_general(
            h, wlin_r[...], (((1,), (1,)), ((), ())),
            precision=jax.lax.Precision.HIGHEST, preferred_element_type=_f32)
        out_r[...] = out + blin_r[...]

    return pl.pallas_call(
        body,
        out_shape=jax.ShapeDtypeStruct((1000, 10), _f32),
    )(a0, a1, c0, c1, x2, wl, bl, wr, wlin, blin)


def kernel(x, edge_index, W_l, b_l, W_r, W_lin, b_lin):
    x_pair = jnp.pad(
        jnp.concatenate([x, jnp.ones_like(x)], axis=1),
        ((0, NPAD - N), (0, PW - 2)))
    zeros = jnp.zeros((NPAD, PW), _f32)
    edge4 = edge_index.reshape(2, NCHUNKS, ROWS, LANES)
    accP = _sc_aggregate(x_pair, zeros, edge4)

    a0 = accP[0, :N, 0].reshape(1000, 100)
    a1 = accP[1, :N, 0].reshape(1000, 100)
    c0 = accP[0, :N, 1].reshape(1000, 100)
    c1 = accP[1, :N, 1].reshape(1000, 100)
    x2 = x.reshape(1000, 100)
    wl = W_l.reshape(1, 4)
    wr = W_r.reshape(1, 4)
    bl = b_l.reshape(1, 4)
    blin = b_lin.reshape(1, 10)
    return _epilogue(a0, a1, c0, c1, x2, wl, bl, wr, W_lin, blin)


# speedup vs baseline: 1.0816x; 1.0099x over previous
import functools

import jax
import jax.numpy as jnp
from jax import lax
from jax.experimental import pallas as pl
from jax.experimental.pallas import tpu as pltpu
from jax.experimental.pallas import tpu_sc as plsc

N = 100000
E = 3200000
NPAD = 102400
RP = NPAD // 16
PW = 8
ROWS = 4
LANES = 256
CHUNK = ROWS * LANES
NCHUNKS = E // CHUNK
NW = 32
ITERS = (NCHUNKS + NW - 1) // NW

_f32 = jnp.float32
_i32 = jnp.int32


def _sc_aggregate(x_pair, zeros, edge4):
    mesh = plsc.VectorSubcoreMesh(core_axis_name="c", subcore_axis_name="s")

    @functools.partial(
        pl.kernel,
        out_type=jax.ShapeDtypeStruct((2, NPAD, PW), _f32),
        mesh=mesh,
        scratch_types=(
            [pltpu.VMEM((LANES,), _i32) for _ in range(8 * ROWS)]
            + [pltpu.VMEM((LANES, PW), _f32) for _ in range(2 * ROWS)]
            + [
                pltpu.VMEM_SHARED((NPAD, PW), _f32),
                pltpu.VMEM_SHARED((NPAD, PW), _f32),
                pltpu.SemaphoreType.DMA,
                pltpu.SemaphoreType.DMA,
                pltpu.SemaphoreType.DMA,
                pltpu.SemaphoreType.DMA,
                pltpu.SemaphoreType.DMA,
                pltpu.SemaphoreType.DMA,
                pltpu.SemaphoreType.DMA,
            ]
        ),
        compiler_params=pltpu.CompilerParams(use_tc_tiling_on_sc=False),
    )
    def sc_agg(xp_hbm, z_hbm, edge_hbm, out_hbm, *refs):
        src_s = [refs[2 * s * ROWS:(2 * s + 1) * ROWS] for s in range(4)]
        dst_s = [refs[(2 * s + 1) * ROWS:(2 * s + 2) * ROWS] for s in range(4)]
        val_s = [refs[(8 + v) * ROWS:(9 + v) * ROWS] for v in range(2)]
        (x_sp, acc_sp, gsem, isem0, isem1, isem2, isem3,
         psem0, psem1) = refs[10 * ROWS:]
        isem_s = [isem0, isem1, isem2, isem3]
        psem_s = [psem0, psem1]
        cid = lax.axis_index("c")
        sid = lax.axis_index("s")
        wid = sid * 2 + cid
        off = sid * RP

        pltpu.sync_copy(xp_hbm.at[pl.ds(off, RP)], x_sp.at[pl.ds(off, RP)])
        pltpu.sync_copy(z_hbm.at[pl.ds(off, RP)], acc_sp.at[pl.ds(off, RP)])

        plsc.subcore_barrier()

        def issue_idx(t, src_v, dst_v, isem):
            for r in range(ROWS):
                pltpu.async_copy(edge_hbm.at[0, t, r], src_v[r], isem)
            for r in range(ROWS):
                pltpu.async_copy(edge_hbm.at[1, t, r], dst_v[r], isem)

        def wait_idx(t, src_v, dst_v, isem):
            for r in range(ROWS):
                pltpu.make_async_copy(edge_hbm.at[0, t, r], src_v[r],
                                      isem).wait()
            for r in range(ROWS):
                pltpu.make_async_copy(edge_hbm.at[1, t, r], dst_v[r],
                                      isem).wait()

        def process(src_v, dst_v, val_v, psem):
            gets = [
                pltpu.async_copy(x_sp.at[src_v[r]], val_v[r], gsem)
                for r in range(ROWS)
            ]
            for r in range(ROWS):
                gets[r].wait()
                pltpu.async_copy(val_v[r], acc_sp.at[dst_v[r]],
                                 psem, add=True)

        def wait_puts(dst_v, val_v, psem):
            for r in range(ROWS):
                pltpu.make_async_copy(val_v[r], acc_sp.at[dst_v[r]],
                                      psem).wait()

        issue_idx(wid, src_s[0], dst_s[0], isem_s[0])
        issue_idx(wid + NW, src_s[1], dst_s[1], isem_s[1])

        def chunk_body(i, _):
            for u in range(4):
                k = 4 * i + u
                t = wid + NW * k
                tp = t + 2 * NW

                @pl.when(t < NCHUNKS)
                def _(u=u, t=t):
                    wait_idx(t, src_s[u], dst_s[u], isem_s[u])
                    process(src_s[u], dst_s[u], val_s[u % 2], psem_s[u % 2])

                up = (u - 1) % 4

                @pl.when(jnp.logical_and(k > 0, t - NW < NCHUNKS))
                def _(up=up):
                    wait_puts(dst_s[up], val_s[up % 2], psem_s[up % 2])

                un = (u + 2) % 4

                @pl.when(tp < NCHUNKS)
                def _(un=un, tp=tp):
                    issue_idx(tp, src_s[un], dst_s[un], isem_s[un])

            return 0

        lax.fori_loop(0, (ITERS + 3) // 4, chunk_body, 0)

        plsc.subcore_barrier()

        pltpu.sync_copy(acc_sp.at[pl.ds(off, RP)],
                        out_hbm.at[cid, pl.ds(off, RP)])

    return sc_agg(x_pair, zeros, edge4)


def _epilogue(a0, a1, c0, c1, x2, wl, bl, wr, wlin, blin):
    def body(a0_r, a1_r, c0_r, c1_r, x_r, wl_r, bl_r, wr_r, wlin_r, blin_r, out_r):
        agg = a0_r[...] + a1_r[...]
        cnt = jnp.maximum(c0_r[...] + c1_r[...], 1.0)
        mean = agg / cnt
        xv = x_r[...]

        kk = lax.broadcasted_iota(_i32, (100, 400), 0)
        jj = lax.broadcasted_iota(_i32, (100, 400), 1)
        f = jj - 4 * (jj // 4)
        sel = (jj // 4) == kk

        def expand(w_r):
            v = jnp.where(
                f == 0, w_r[0, 0],
                jnp.where(f == 1, w_r[0, 1],
                          jnp.where(f == 2, w_r[0, 2], w_r[0, 3])))
            return jnp.where(sel, v, 0.0)

        s_l = expand(wl_r)
        s_r = expand(wr_r)

        j2 = lax.broadcasted_iota(_i32, (8, 400), 1)
        f2 = j2 - 4 * (j2 // 4)
        brow = jnp.where(
            f2 == 0, bl_r[0, 0],
            jnp.where(f2 == 1, bl_r[0, 1],
                      jnp.where(f2 == 2, bl_r[0, 2], bl_r[0, 3])))[:1]

        h = (jax.lax.dot(mean, s_l, precision=jax.lax.Precision.HIGHEST,
                         preferred_element_type=_f32)
             + jax.lax.dot(xv, s_r, precision=jax.lax.Precision.HIGHEST,
                           preferred_element_type=_f32))
        h = jnp.maximum(h + brow, 0.0)
        out = jax.lax.dot_general(
            h, wlin_r[...], (((1,), (1,)), ((), ())),
            precision=jax.lax.Precision.HIGHEST, preferred_element_type=_f32)
        out_r[...] = out + blin_r[...]

    return pl.pallas_call(
        body,
        out_shape=jax.ShapeDtypeStruct((1000, 10), _f32),
    )(a0, a1, c0, c1, x2, wl, bl, wr, wlin, blin)


def kernel(x, edge_index, W_l, b_l, W_r, W_lin, b_lin):
    x_pair = jnp.pad(
        jnp.concatenate([x, jnp.ones_like(x)], axis=1),
        ((0, NPAD - N), (0, PW - 2)))
    zeros = jnp.zeros((NPAD, PW), _f32)
    edge4 = edge_index.reshape(2, NCHUNKS, ROWS, LANES)
    accP = _sc_aggregate(x_pair, zeros, edge4)

    a0 = accP[0, :N, 0].reshape(1000, 100)
    a1 = accP[1, :N, 0].reshape(1000, 100)
    c0 = accP[0, :N, 1].reshape(1000, 100)
    c1 = accP[1, :N, 1].reshape(1000, 100)
    x2 = x.reshape(1000, 100)
    wl = W_l.reshape(1, 4)
    wr = W_r.reshape(1, 4)
    bl = b_l.reshape(1, 4)
    blin = b_lin.reshape(1, 10)
    return _epilogue(a0, a1, c0, c1, x2, wl, bl, wr, W_lin, blin)


# speedup vs baseline: 1.0831x; 1.0014x over previous
import functools

import jax
import jax.numpy as jnp
from jax import lax
from jax.experimental import pallas as pl
from jax.experimental.pallas import tpu as pltpu
from jax.experimental.pallas import tpu_sc as plsc

N = 100000
E = 3200000
NPAD = 102400
RP = NPAD // 16
PW = 8
ROWS = 2
LANES = 512
CHUNK = ROWS * LANES
NCHUNKS = E // CHUNK
NW = 32
ITERS = (NCHUNKS + NW - 1) // NW

_f32 = jnp.float32
_i32 = jnp.int32


def _sc_aggregate(x_pair, zeros, edge4):
    mesh = plsc.VectorSubcoreMesh(core_axis_name="c", subcore_axis_name="s")

    @functools.partial(
        pl.kernel,
        out_type=jax.ShapeDtypeStruct((2, NPAD, PW), _f32),
        mesh=mesh,
        scratch_types=(
            [pltpu.VMEM((LANES,), _i32) for _ in range(8 * ROWS)]
            + [pltpu.VMEM((LANES, PW), _f32) for _ in range(2 * ROWS)]
            + [
                pltpu.VMEM_SHARED((NPAD, PW), _f32),
                pltpu.VMEM_SHARED((NPAD, PW), _f32),
                pltpu.SemaphoreType.DMA,
                pltpu.SemaphoreType.DMA,
                pltpu.SemaphoreType.DMA,
                pltpu.SemaphoreType.DMA,
                pltpu.SemaphoreType.DMA,
                pltpu.SemaphoreType.DMA,
                pltpu.SemaphoreType.DMA,
            ]
        ),
        compiler_params=pltpu.CompilerParams(use_tc_tiling_on_sc=False),
    )
    def sc_agg(xp_hbm, z_hbm, edge_hbm, out_hbm, *refs):
        src_s = [refs[2 * s * ROWS:(2 * s + 1) * ROWS] for s in range(4)]
        dst_s = [refs[(2 * s + 1) * ROWS:(2 * s + 2) * ROWS] for s in range(4)]
        val_s = [refs[(8 + v) * ROWS:(9 + v) * ROWS] for v in range(2)]
        (x_sp, acc_sp, gsem, isem0, isem1, isem2, isem3,
         psem0, psem1) = refs[10 * ROWS:]
        isem_s = [isem0, isem1, isem2, isem3]
        psem_s = [psem0, psem1]
        cid = lax.axis_index("c")
        sid = lax.axis_index("s")
        wid = sid * 2 + cid
        off = sid * RP

        pltpu.sync_copy(xp_hbm.at[pl.ds(off, RP)], x_sp.at[pl.ds(off, RP)])
        pltpu.sync_copy(z_hbm.at[pl.ds(off, RP)], acc_sp.at[pl.ds(off, RP)])

        plsc.subcore_barrier()

        def issue_idx(t, src_v, dst_v, isem):
            for r in range(ROWS):
                pltpu.async_copy(edge_hbm.at[0, t, r], src_v[r], isem)
            for r in range(ROWS):
                pltpu.async_copy(edge_hbm.at[1, t, r], dst_v[r], isem)

        def wait_idx(t, src_v, dst_v, isem):
            for r in range(ROWS):
                pltpu.make_async_copy(edge_hbm.at[0, t, r], src_v[r],
                                      isem).wait()
            for r in range(ROWS):
                pltpu.make_async_copy(edge_hbm.at[1, t, r], dst_v[r],
                                      isem).wait()

        def process(src_v, dst_v, val_v, psem):
            gets = [
                pltpu.async_copy(x_sp.at[src_v[r]], val_v[r], gsem)
                for r in range(ROWS)
            ]
            for r in range(ROWS):
                gets[r].wait()
                pltpu.async_copy(val_v[r], acc_sp.at[dst_v[r]],
                                 psem, add=True)

        def wait_puts(dst_v, val_v, psem):
            for r in range(ROWS):
                pltpu.make_async_copy(val_v[r], acc_sp.at[dst_v[r]],
                                      psem).wait()

        issue_idx(wid, src_s[0], dst_s[0], isem_s[0])
        issue_idx(wid + NW, src_s[1], dst_s[1], isem_s[1])

        def chunk_body(i, _):
            for u in range(4):
                k = 4 * i + u
                t = wid + NW * k
                tp = t + 2 * NW

                @pl.when(t < NCHUNKS)
                def _(u=u, t=t):
                    wait_idx(t, src_s[u], dst_s[u], isem_s[u])
                    process(src_s[u], dst_s[u], val_s[u % 2], psem_s[u % 2])

                up = (u - 1) % 4

                @pl.when(jnp.logical_and(k > 0, t - NW < NCHUNKS))
                def _(up=up):
                    wait_puts(dst_s[up], val_s[up % 2], psem_s[up % 2])

                un = (u + 2) % 4

                @pl.when(tp < NCHUNKS)
                def _(un=un, tp=tp):
                    issue_idx(tp, src_s[un], dst_s[un], isem_s[un])

            return 0

        lax.fori_loop(0, (ITERS + 3) // 4, chunk_body, 0)

        plsc.subcore_barrier()

        pltpu.sync_copy(acc_sp.at[pl.ds(off, RP)],
                        out_hbm.at[cid, pl.ds(off, RP)])

    return sc_agg(x_pair, zeros, edge4)


def _epilogue(a0, a1, c0, c1, x2, wl, bl, wr, wlin, blin):
    def body(a0_r, a1_r, c0_r, c1_r, x_r, wl_r, bl_r, wr_r, wlin_r, blin_r, out_r):
        agg = a0_r[...] + a1_r[...]
        cnt = jnp.maximum(c0_r[...] + c1_r[...], 1.0)
        mean = agg / cnt
        xv = x_r[...]

        kk = lax.broadcasted_iota(_i32, (100, 400), 0)
        jj = lax.broadcasted_iota(_i32, (100, 400), 1)
        f = jj - 4 * (jj // 4)
        sel = (jj // 4) == kk

        def expand(w_r):
            v = jnp.where(
                f == 0, w_r[0, 0],
                jnp.where(f == 1, w_r[0, 1],
                          jnp.where(f == 2, w_r[0, 2], w_r[0, 3])))
            return jnp.where(sel, v, 0.0)

        s_l = expand(wl_r)
        s_r = expand(wr_r)

        j2 = lax.broadcasted_iota(_i32, (8, 400), 1)
        f2 = j2 - 4 * (j2 // 4)
        brow = jnp.where(
            f2 == 0, bl_r[0, 0],
            jnp.where(f2 == 1, bl_r[0, 1],
                      jnp.where(f2 == 2, bl_r[0, 2], bl_r[0, 3])))[:1]

        h = (jax.lax.dot(mean, s_l, precision=jax.lax.Precision.HIGHEST,
                         preferred_element_type=_f32)
             + jax.lax.dot(xv, s_r, precision=jax.lax.Precision.HIGHEST,
                           preferred_element_type=_f32))
        h = jnp.maximum(h + brow, 0.0)
        out = jax.lax.dot_general(
            h, wlin_r[...], (((1,), (1,)), ((), ())),
            precision=jax.lax.Precision.HIGHEST, preferred_element_type=_f32)
        out_r[...] = out + blin_r[...]

    return pl.pallas_call(
        body,
        out_shape=jax.ShapeDtypeStruct((1000, 10), _f32),
    )(a0, a1, c0, c1, x2, wl, bl, wr, wlin, blin)


def kernel(x, edge_index, W_l, b_l, W_r, W_lin, b_lin):
    x_pair = jnp.pad(
        jnp.concatenate([x, jnp.ones_like(x)], axis=1),
        ((0, NPAD - N), (0, PW - 2)))
    zeros = jnp.zeros((NPAD, PW), _f32)
    edge4 = edge_index.reshape(2, NCHUNKS, ROWS, LANES)
    accP = _sc_aggregate(x_pair, zeros, edge4)

    a0 = accP[0, :N, 0].reshape(1000, 100)
    a1 = accP[1, :N, 0].reshape(1000, 100)
    c0 = accP[0, :N, 1].reshape(1000, 100)
    c1 = accP[1, :N, 1].reshape(1000, 100)
    x2 = x.reshape(1000, 100)
    wl = W_l.reshape(1, 4)
    wr = W_r.reshape(1, 4)
    bl = b_l.reshape(1, 4)
    blin = b_lin.reshape(1, 10)
    return _epilogue(a0, a1, c0, c1, x2, wl, bl, wr, W_lin, blin)
